# baseline (speedup 1.0000x reference)
import jax
import jax.numpy as jnp
import numpy as np
from jax import lax
from jax.experimental import pallas as pl
from jax.experimental.pallas import tpu as pltpu

HAND_TYPE_COUNT = 13
D_MODEL = 64
BB = 32
NB = 3

_S = np.zeros((3, 16), dtype=np.float32)
_S[0, :13] = 1.0
_S[1, 13] = 1.0
_S[2, 14] = 1.0


def _compute_block(hl, s, tab, gamma, beta):
    bb, n, _ = hl.shape
    tok = bb * n
    hl2 = hl.reshape(tok, 3)
    t = jnp.dot(hl2, s, preferred_element_type=jnp.float32)
    col = lax.broadcasted_iota(jnp.int32, (tok, 16), 1)
    ti = t.astype(jnp.int32)
    m = jnp.where(
        col < 13,
        (ti == col).astype(jnp.float32),
        jnp.where(col < 15, t, 1.0),
    )
    x = jnp.dot(m, tab, preferred_element_type=jnp.float32)
    mu = jnp.mean(x, axis=-1, keepdims=True)
    xc = x - mu
    var = jnp.mean(xc * xc, axis=-1, keepdims=True)
    xn = xc * lax.rsqrt(var + 1e-5)
    y = xn * gamma + beta
    return y.reshape(bb, n, D_MODEL)


def _pipelined_kernel(
    hl_hbm, s_ref, tab_ref, gamma_ref, beta_ref, out_hbm, inb, outb, insem, outsem
):
    G = hl_hbm.shape[0] // BB

    def in_copy(i, slot):
        return pltpu.make_async_copy(
            hl_hbm.at[pl.ds(i * BB, BB)], inb.at[slot], insem.at[slot]
        )

    def out_copy(i, slot):
        return pltpu.make_async_copy(
            outb.at[slot], out_hbm.at[pl.ds(i * BB, BB)], outsem.at[slot]
        )

    in_copy(0, 0).start()
    in_copy(1, 1).start()
    in_copy(2, 2).start()

    def step(i, carry):
        slot = lax.rem(i, NB)

        @pl.when(i >= NB)
        def _():
            out_copy(i - NB, slot).wait()

        in_copy(i, slot).wait()
        y = _compute_block(
            inb.at[slot][...],
            s_ref[...],
            tab_ref[...],
            gamma_ref[...],
            beta_ref[...],
        )
        outb.at[slot][...] = y
        out_copy(i, slot).start()

        @pl.when(i + NB < G)
        def _():
            in_copy(i + NB, slot).start()

        return carry

    lax.fori_loop(0, G, step, 0)
    out_copy(G - 3, lax.rem(G - 3, NB)).wait()
    out_copy(G - 2, lax.rem(G - 2, NB)).wait()
    out_copy(G - 1, lax.rem(G - 1, NB)).wait()


def kernel(hand_levels, type_emb, W, b, gamma, beta):
    B, N, _ = hand_levels.shape
    tab = jnp.concatenate(
        [type_emb, W, b[None, :].astype(jnp.float32)], axis=0
    )
    out = pl.pallas_call(
        _pipelined_kernel,
        in_specs=[
            pl.BlockSpec(memory_space=pl.ANY),
            pl.BlockSpec(memory_space=pltpu.VMEM),
            pl.BlockSpec(memory_space=pltpu.VMEM),
            pl.BlockSpec(memory_space=pltpu.VMEM),
            pl.BlockSpec(memory_space=pltpu.VMEM),
        ],
        out_specs=pl.BlockSpec(memory_space=pl.ANY),
        out_shape=jax.ShapeDtypeStruct((B, N, D_MODEL), jnp.float32),
        scratch_shapes=[
            pltpu.VMEM((NB, BB, N, 3), jnp.float32),
            pltpu.VMEM((NB, BB, N, D_MODEL), jnp.float32),
            pltpu.SemaphoreType.DMA((NB,)),
            pltpu.SemaphoreType.DMA((NB,)),
        ],
    )(
        hand_levels,
        jnp.asarray(_S),
        tab,
        gamma.reshape(1, D_MODEL),
        beta.reshape(1, D_MODEL),
    )
    return out


# speedup vs baseline: 1.0208x; 1.0208x over previous
import jax
import jax.numpy as jnp
import numpy as np
from jax import lax
from jax.experimental import pallas as pl
from jax.experimental.pallas import tpu as pltpu

HAND_TYPE_COUNT = 13
D_MODEL = 64
BB = 64
NB = 3

_S = np.zeros((3, 16), dtype=np.float32)
_S[0, :13] = 1.0
_S[1, 13] = 1.0
_S[2, 14] = 1.0


def _compute_block(hl, s, tab, gamma, beta):
    bb, n, _ = hl.shape
    tok = bb * n
    hl2 = hl.reshape(tok, 3)
    t = jnp.dot(hl2, s, preferred_element_type=jnp.float32)
    col = lax.broadcasted_iota(jnp.int32, (tok, 16), 1)
    ti = t.astype(jnp.int32)
    m = jnp.where(
        col < 13,
        (ti == col).astype(jnp.float32),
        jnp.where(col < 15, t, 1.0),
    )
    x = jnp.dot(m, tab, preferred_element_type=jnp.float32)
    mu = jnp.mean(x, axis=-1, keepdims=True)
    xc = x - mu
    var = jnp.mean(xc * xc, axis=-1, keepdims=True)
    xn = xc * lax.rsqrt(var + 1e-5)
    y = xn * gamma + beta
    return y.reshape(bb, n, D_MODEL)


def _pipelined_kernel(
    hl_hbm, s_ref, tab_ref, gamma_ref, beta_ref, out_hbm, inb, outb, insem, outsem
):
    G = hl_hbm.shape[0] // BB

    def in_copy(i, slot):
        return pltpu.make_async_copy(
            hl_hbm.at[pl.ds(i * BB, BB)], inb.at[slot], insem.at[slot]
        )

    def out_copy(i, slot):
        return pltpu.make_async_copy(
            outb.at[slot], out_hbm.at[pl.ds(i * BB, BB)], outsem.at[slot]
        )

    in_copy(0, 0).start()
    in_copy(1, 1).start()
    in_copy(2, 2).start()

    def step(i, carry):
        slot = lax.rem(i, NB)

        @pl.when(i >= NB)
        def _():
            out_copy(i - NB, slot).wait()

        in_copy(i, slot).wait()
        y = _compute_block(
            inb.at[slot][...],
            s_ref[...],
            tab_ref[...],
            gamma_ref[...],
            beta_ref[...],
        )
        outb.at[slot][...] = y
        out_copy(i, slot).start()

        @pl.when(i + NB < G)
        def _():
            in_copy(i + NB, slot).start()

        return carry

    lax.fori_loop(0, G, step, 0)
    out_copy(G - 3, lax.rem(G - 3, NB)).wait()
    out_copy(G - 2, lax.rem(G - 2, NB)).wait()
    out_copy(G - 1, lax.rem(G - 1, NB)).wait()


def kernel(hand_levels, type_emb, W, b, gamma, beta):
    B, N, _ = hand_levels.shape
    tab = jnp.concatenate(
        [type_emb, W, b[None, :].astype(jnp.float32)], axis=0
    )
    out = pl.pallas_call(
        _pipelined_kernel,
        in_specs=[
            pl.BlockSpec(memory_space=pl.ANY),
            pl.BlockSpec(memory_space=pltpu.VMEM),
            pl.BlockSpec(memory_space=pltpu.VMEM),
            pl.BlockSpec(memory_space=pltpu.VMEM),
            pl.BlockSpec(memory_space=pltpu.VMEM),
        ],
        out_specs=pl.BlockSpec(memory_space=pl.ANY),
        out_shape=jax.ShapeDtypeStruct((B, N, D_MODEL), jnp.float32),
        scratch_shapes=[
            pltpu.VMEM((NB, BB, N, 3), jnp.float32),
            pltpu.VMEM((NB, BB, N, D_MODEL), jnp.float32),
            pltpu.SemaphoreType.DMA((NB,)),
            pltpu.SemaphoreType.DMA((NB,)),
        ],
    )(
        hand_levels,
        jnp.asarray(_S),
        tab,
        gamma.reshape(1, D_MODEL),
        beta.reshape(1, D_MODEL),
    )
    return out
